# baseline (speedup 1.0000x reference)
import functools

import jax
import jax.numpy as jnp
from jax import lax
from jax.experimental import pallas as pl
from jax.experimental.pallas import tpu as pltpu
from jax.experimental.pallas import tpu_sc as plsc

VOCAB = 1000000
DIM = 64
DPAD = 128
B = 4096
L = 200
N_TOTAL = B * L

NC = 2
NS = 16
NW = NC * NS

TC_COLS = 512
TAIL_VOCAB = 999936
N_CHUNKS = TAIL_VOCAB // TC_COLS
P_OUTER = -(-N_CHUNKS // NW)

PER_W = N_TOTAL // NW
RCH = 2
G = RCH * L
N_OUTER = PER_W // G
SPLIT = 128
REST = L - SPLIT

_mesh = plsc.VectorSubcoreMesh(core_axis_name="c", subcore_axis_name="s")


@functools.partial(
    pl.kernel,
    mesh=_mesh,
    out_type=jax.ShapeDtypeStruct((VOCAB // 2, DPAD), jnp.float32),
    scratch_types=[
        pltpu.VMEM((DIM, TC_COLS), jnp.float32),
        pltpu.VMEM((TC_COLS // 2, DPAD), jnp.float32),
        pltpu.VMEM((32, DPAD), jnp.float32),
    ],
    compiler_params=pltpu.CompilerParams(needs_layout_passes=False),
)
def _pack_table(tableT_hbm, tail_hbm, packed_hbm, tin, tout, ttail):
    wid = lax.axis_index("s") * NC + lax.axis_index("c")
    lane = lax.iota(jnp.int32, 16)

    @pl.when(wid == 0)
    def _():
        pltpu.sync_copy(tail_hbm, ttail)
        pltpu.sync_copy(ttail, packed_hbm.at[pl.ds(TAIL_VOCAB // 2, 32)])

    def chunk(g, carry):
        cid = g * NW + wid

        @pl.when(cid < N_CHUNKS)
        def _():
            c0 = pl.multiple_of(cid * TC_COLS, 128)
            pltpu.sync_copy(tableT_hbm.at[:, pl.ds(c0, TC_COLS)], tin)

            def prow(p, carry):
                for j in range(8):
                    dvec = 16 * j + lane if j < 4 else 16 * j - DIM + lane
                    col = jnp.where(j < 4, 2 * p, 2 * p + 1)
                    vec = plsc.load_gather(
                        tin, [dvec, jnp.full((16,), 0, jnp.int32) + col]
                    )
                    tout[p, pl.ds(16 * j, 16)] = vec
                return carry

            lax.fori_loop(0, TC_COLS // 2, prow, 0)
            p0 = pl.multiple_of(lax.shift_right_logical(c0, 1), 8)
            pltpu.sync_copy(tout, packed_hbm.at[pl.ds(p0, TC_COLS // 2)])

        return carry

    lax.fori_loop(0, P_OUTER, chunk, 0)


@functools.partial(
    pl.kernel,
    mesh=_mesh,
    out_type=jax.ShapeDtypeStruct((B, L, DPAD), jnp.float32),
    scratch_types=[
        pltpu.VMEM((RCH, L), jnp.int32),
        pltpu.VMEM((RCH, L, DIM), jnp.float32),
        pltpu.SemaphoreType.DMA,
    ],
    compiler_params=pltpu.CompilerParams(use_tc_tiling_on_sc=False),
)
def _emb_lookup(x_hbm, table_hbm, out_hbm, idx_v, rows_v, sem):
    wid = lax.axis_index("s") * NC + lax.axis_index("c")
    row0 = wid * (B // NW)

    def chunk(c, carry):
        b0 = row0 + c * RCH
        pltpu.sync_copy(x_hbm.at[pl.ds(b0, RCH)], idx_v)
        cps = []
        for j in range(RCH):
            cps.append(pltpu.async_copy(
                table_hbm.at[idx_v.at[j, pl.ds(0, SPLIT)]],
                rows_v.at[j, pl.ds(0, SPLIT)],
                sem,
            ))
            cps.append(pltpu.async_copy(
                table_hbm.at[idx_v.at[j, pl.ds(SPLIT, REST)]],
                rows_v.at[j, pl.ds(SPLIT, REST)],
                sem,
            ))
        for cp in cps:
            cp.wait()
        pltpu.sync_copy(
            rows_v,
            out_hbm.at[pl.ds(b0, RCH), :, pl.ds(0, DIM)],
        )
        return carry

    lax.fori_loop(0, N_OUTER, chunk, 0)


def kernel(x, table):
    tableT = table.T
    tail = table[TAIL_VOCAB:].reshape(32, DPAD)
    packed = _pack_table(tableT, tail)
    rowmajor = packed.reshape(VOCAB, DIM)
    out128 = _emb_lookup(x, rowmajor)
    return out128[:, :, :DIM]


# speedup vs baseline: 1.1534x; 1.1534x over previous
import functools

import jax
import jax.numpy as jnp
from jax import lax
from jax.experimental import pallas as pl
from jax.experimental.pallas import tpu as pltpu
from jax.experimental.pallas import tpu_sc as plsc

VOCAB = 1000000
DIM = 64
DPAD = 128
B = 4096
L = 200
N_TOTAL = B * L

NC = 2
NS = 16
NW = NC * NS

TC_COLS = 512
TAIL_VOCAB = 999936
N_CHUNKS = TAIL_VOCAB // TC_COLS
P_OUTER = -(-N_CHUNKS // NW)

PER_W = N_TOTAL // NW
RCH = 2
G = RCH * L
N_OUTER = PER_W // G
SPLIT = 128
REST = L - SPLIT

_mesh = plsc.VectorSubcoreMesh(core_axis_name="c", subcore_axis_name="s")


@functools.partial(
    pl.kernel,
    mesh=_mesh,
    out_type=jax.ShapeDtypeStruct((VOCAB // 2, DPAD), jnp.float32),
    scratch_types=[
        pltpu.VMEM((DIM, TC_COLS), jnp.float32),
        pltpu.VMEM((TC_COLS // 2, DPAD), jnp.float32),
        pltpu.VMEM((32, DPAD), jnp.float32),
    ],
    compiler_params=pltpu.CompilerParams(needs_layout_passes=False),
)
def _pack_table(tableT_hbm, tail_hbm, packed_hbm, tin, tout, ttail):
    wid = lax.axis_index("s") * NC + lax.axis_index("c")
    lane = lax.iota(jnp.int32, 16)

    @pl.when(wid == 0)
    def _():
        pltpu.sync_copy(tail_hbm, ttail)
        pltpu.sync_copy(ttail, packed_hbm.at[pl.ds(TAIL_VOCAB // 2, 32)])

    def chunk(g, carry):
        cid = g * NW + wid

        @pl.when(cid < N_CHUNKS)
        def _():
            c0 = pl.multiple_of(cid * TC_COLS, 128)
            pltpu.sync_copy(tableT_hbm.at[:, pl.ds(c0, TC_COLS)], tin)
            pcb = (lane & 1) * DIM

            def kloop(k, carry):
                prow = lax.shift_right_logical(16 * k + lane, 1)
                for d in range(DIM):
                    vec = tin[d, pl.ds(pl.multiple_of(16 * k, 16), 16)]
                    plsc.store_scatter(tout, [prow, pcb + d], vec)
                return carry

            lax.fori_loop(0, TC_COLS // 16, kloop, 0)
            p0 = pl.multiple_of(lax.shift_right_logical(c0, 1), 8)
            pltpu.sync_copy(tout, packed_hbm.at[pl.ds(p0, TC_COLS // 2)])

        return carry

    lax.fori_loop(0, P_OUTER, chunk, 0)


@functools.partial(
    pl.kernel,
    mesh=_mesh,
    out_type=jax.ShapeDtypeStruct((B, L, DPAD), jnp.float32),
    scratch_types=[
        pltpu.VMEM((RCH, L), jnp.int32),
        pltpu.VMEM((RCH, L, DIM), jnp.float32),
        pltpu.SemaphoreType.DMA,
    ],
    compiler_params=pltpu.CompilerParams(use_tc_tiling_on_sc=False),
)
def _emb_lookup(x_hbm, table_hbm, out_hbm, idx_v, rows_v, sem):
    wid = lax.axis_index("s") * NC + lax.axis_index("c")
    row0 = wid * (B // NW)

    def chunk(c, carry):
        b0 = row0 + c * RCH
        pltpu.sync_copy(x_hbm.at[pl.ds(b0, RCH)], idx_v)
        cps = []
        for j in range(RCH):
            cps.append(pltpu.async_copy(
                table_hbm.at[idx_v.at[j, pl.ds(0, SPLIT)]],
                rows_v.at[j, pl.ds(0, SPLIT)],
                sem,
            ))
            cps.append(pltpu.async_copy(
                table_hbm.at[idx_v.at[j, pl.ds(SPLIT, REST)]],
                rows_v.at[j, pl.ds(SPLIT, REST)],
                sem,
            ))
        for cp in cps:
            cp.wait()
        pltpu.sync_copy(
            rows_v,
            out_hbm.at[pl.ds(b0, RCH), :, pl.ds(0, DIM)],
        )
        return carry

    lax.fori_loop(0, N_OUTER, chunk, 0)


def kernel(x, table):
    tableT = table.T
    tail = table[TAIL_VOCAB:].reshape(32, DPAD)
    packed = _pack_table(tableT, tail)
    rowmajor = packed.reshape(VOCAB, DIM)
    out128 = _emb_lookup(x, rowmajor)
    return out128[:, :, :DIM]


# speedup vs baseline: 2.1434x; 1.8584x over previous
import functools

import jax
import jax.numpy as jnp
from jax import lax
from jax.experimental import pallas as pl
from jax.experimental.pallas import tpu as pltpu
from jax.experimental.pallas import tpu_sc as plsc

VOCAB = 1000000
DIM = 64
DPAD = 128
B = 4096
L = 200
N_TOTAL = B * L

NC = 2
NS = 16
NW = NC * NS

TC_COLS = 512
TAIL_VOCAB = 999936
N_CHUNKS = TAIL_VOCAB // TC_COLS
P_OUTER = -(-N_CHUNKS // NW)

PER_W = N_TOTAL // NW
RCH = 2
G = RCH * L
N_OUTER = PER_W // G
SPLIT = 128
REST = L - SPLIT

_mesh = plsc.VectorSubcoreMesh(core_axis_name="c", subcore_axis_name="s")


@functools.partial(
    pl.kernel,
    mesh=_mesh,
    out_type=jax.ShapeDtypeStruct((VOCAB // 2, DPAD), jnp.float32),
    scratch_types=[
        pltpu.VMEM((DIM, TC_COLS), jnp.float32),
        pltpu.VMEM((TC_COLS // 2, DPAD), jnp.float32),
        pltpu.VMEM((32, DPAD), jnp.float32),
    ],
    compiler_params=pltpu.CompilerParams(needs_layout_passes=False),
)
def _pack_table(tableT_hbm, tail_hbm, packed_hbm, tin, tout, ttail):
    wid = lax.axis_index("s") * NC + lax.axis_index("c")
    lane = lax.iota(jnp.int32, 16)

    @pl.when(wid == 0)
    def _():
        pltpu.sync_copy(tail_hbm, ttail)
        pltpu.sync_copy(ttail, packed_hbm.at[pl.ds(TAIL_VOCAB // 2, 32)])

    def chunk(g, carry):
        cid = g * NW + wid

        @pl.when(cid < N_CHUNKS)
        def _():
            c0 = pl.multiple_of(cid * TC_COLS, 128)
            pltpu.sync_copy(tableT_hbm.at[:, pl.ds(c0, TC_COLS)], tin)
            pcb = (lane & 1) * DIM

            def kloop(k, carry):
                prow = lax.shift_right_logical(16 * k + lane, 1)
                for d in range(DIM):
                    vec = tin[d, pl.ds(pl.multiple_of(16 * k, 16), 16)]
                    plsc.store_scatter(tout, [prow, pcb + d], vec)
                return carry

            lax.fori_loop(0, TC_COLS // 16, kloop, 0)
            p0 = pl.multiple_of(lax.shift_right_logical(c0, 1), 8)
            pltpu.sync_copy(tout, packed_hbm.at[pl.ds(p0, TC_COLS // 2)])

        return carry

    lax.fori_loop(0, P_OUTER, chunk, 0)


@functools.partial(
    pl.kernel,
    mesh=_mesh,
    out_type=jax.ShapeDtypeStruct((B, L, DPAD), jnp.float32),
    scratch_types=[
        pltpu.VMEM((RCH, L), jnp.int32),
        pltpu.VMEM((RCH, L, DIM), jnp.float32),
        pltpu.SemaphoreType.DMA,
    ],
    compiler_params=pltpu.CompilerParams(use_tc_tiling_on_sc=False),
)
def _emb_lookup(x_hbm, table_hbm, out_hbm, idx_v, rows_v, sem):
    wid = lax.axis_index("s") * NC + lax.axis_index("c")
    row0 = wid * (B // NW)

    def chunk(c, carry):
        b0 = row0 + c * RCH
        pltpu.sync_copy(x_hbm.at[pl.ds(b0, RCH)], idx_v)
        cps = []
        for j in range(RCH):
            cps.append(pltpu.async_copy(
                table_hbm.at[idx_v.at[j, pl.ds(0, SPLIT)]],
                rows_v.at[j, pl.ds(0, SPLIT)],
                sem,
            ))
            cps.append(pltpu.async_copy(
                table_hbm.at[idx_v.at[j, pl.ds(SPLIT, REST)]],
                rows_v.at[j, pl.ds(SPLIT, REST)],
                sem,
            ))
        for cp in cps:
            cp.wait()
        pltpu.sync_copy(
            rows_v,
            out_hbm.at[pl.ds(b0, RCH), :, pl.ds(0, DIM)],
        )
        return carry

    lax.fori_loop(0, N_OUTER, chunk, 0)


def kernel(x, table):
    out128 = _emb_lookup(x, table)
    return out128[:, :, :DIM]


# speedup vs baseline: 2.2796x; 1.0635x over previous
import functools

import jax
import jax.numpy as jnp
from jax import lax
from jax.experimental import pallas as pl
from jax.experimental.pallas import tpu as pltpu
from jax.experimental.pallas import tpu_sc as plsc

VOCAB = 1000000
DIM = 64
DPAD = 128
B = 4096
L = 200

NC = 2
NS = 16
NW = NC * NS

ROWS_W = B // NW
RCH = 4
N_OUTER = ROWS_W // RCH
SPLIT = 128
REST = L - SPLIT

_mesh = plsc.VectorSubcoreMesh(core_axis_name="c", subcore_axis_name="s")


@functools.partial(
    pl.kernel,
    mesh=_mesh,
    out_type=jax.ShapeDtypeStruct((B, L, DPAD), jnp.float32),
    scratch_types=[
        pltpu.VMEM((RCH, L), jnp.int32),
        pltpu.VMEM((RCH, L, DIM), jnp.float32),
        pltpu.VMEM((RCH, L, DIM), jnp.float32),
        pltpu.SemaphoreType.DMA,
        pltpu.SemaphoreType.DMA,
        pltpu.SemaphoreType.DMA,
    ],
    compiler_params=pltpu.CompilerParams(use_tc_tiling_on_sc=False),
)
def _emb_lookup(x_hbm, table_hbm, out_hbm, idx_v, rows0, rows1, gsem, osem0, osem1):
    wid = lax.axis_index("s") * NC + lax.axis_index("c")
    row0 = wid * ROWS_W
    bufs = [(rows0, osem0), (rows1, osem1)]

    def out_window(c):
        return out_hbm.at[pl.ds(row0 + c * RCH, RCH), :, pl.ds(0, DIM)]

    def do_chunk(c, rows_v, osem, first):
        b0 = row0 + c * RCH
        pltpu.sync_copy(x_hbm.at[pl.ds(b0, RCH)], idx_v)
        if not first:
            pltpu.make_async_copy(rows_v, out_window(c - 2), osem).wait()
        cps = []
        for j in range(RCH):
            cps.append(pltpu.async_copy(
                table_hbm.at[idx_v.at[j, pl.ds(0, SPLIT)]],
                rows_v.at[j, pl.ds(0, SPLIT)],
                gsem,
            ))
            cps.append(pltpu.async_copy(
                table_hbm.at[idx_v.at[j, pl.ds(SPLIT, REST)]],
                rows_v.at[j, pl.ds(SPLIT, REST)],
                gsem,
            ))
        for cp in cps:
            cp.wait()
        pltpu.async_copy(rows_v, out_window(c), osem)

    for b in range(2):
        do_chunk(b, *bufs[b], True)

    def super_step(s, carry):
        for b in range(2):
            do_chunk(2 * s + b, *bufs[b], False)
        return carry

    lax.fori_loop(1, N_OUTER // 2, super_step, 0)
    for b in range(2):
        rows_v, osem = bufs[b]
        pltpu.make_async_copy(rows_v, out_window(N_OUTER - 2 + b), osem).wait()


def kernel(x, table):
    out128 = _emb_lookup(x, table)
    return out128[:, :, :DIM]


# speedup vs baseline: 2.2979x; 1.0081x over previous
import functools

import jax
import jax.numpy as jnp
from jax import lax
from jax.experimental import pallas as pl
from jax.experimental.pallas import tpu as pltpu
from jax.experimental.pallas import tpu_sc as plsc

VOCAB = 1000000
DIM = 64
DPAD = 128
B = 4096
L = 200

NC = 2
NS = 16
NW = NC * NS

ROWS_W = B // NW
RCH = 4
N_OUTER = ROWS_W // RCH
SPLIT = 128
REST = L - SPLIT

_mesh = plsc.VectorSubcoreMesh(core_axis_name="c", subcore_axis_name="s")


@functools.partial(
    pl.kernel,
    mesh=_mesh,
    out_type=jax.ShapeDtypeStruct((B, L, DPAD), jnp.float32),
    scratch_types=[
        pltpu.VMEM((ROWS_W, L), jnp.int32),
        pltpu.VMEM((RCH, L, DIM), jnp.float32),
        pltpu.VMEM((RCH, L, DIM), jnp.float32),
        pltpu.SemaphoreType.DMA,
        pltpu.SemaphoreType.DMA,
        pltpu.SemaphoreType.DMA,
    ],
    compiler_params=pltpu.CompilerParams(use_tc_tiling_on_sc=False),
)
def _emb_lookup(x_hbm, table_hbm, out_hbm, idx_v, rows0, rows1, gsem, osem0, osem1):
    wid = lax.axis_index("s") * NC + lax.axis_index("c")
    row0 = wid * ROWS_W
    bufs = [(rows0, osem0), (rows1, osem1)]
    pltpu.sync_copy(x_hbm.at[pl.ds(row0, ROWS_W)], idx_v)

    def out_window(c):
        return out_hbm.at[pl.ds(row0 + c * RCH, RCH), :, pl.ds(0, DIM)]

    def do_chunk(c, rows_v, osem, first):
        if not first:
            pltpu.make_async_copy(rows_v, out_window(c - 2), osem).wait()
        cps = []
        for j in range(RCH):
            cps.append(pltpu.async_copy(
                table_hbm.at[idx_v.at[c * RCH + j, pl.ds(0, SPLIT)]],
                rows_v.at[j, pl.ds(0, SPLIT)],
                gsem,
            ))
            cps.append(pltpu.async_copy(
                table_hbm.at[idx_v.at[c * RCH + j, pl.ds(SPLIT, REST)]],
                rows_v.at[j, pl.ds(SPLIT, REST)],
                gsem,
            ))
        for cp in cps:
            cp.wait()
        pltpu.async_copy(rows_v, out_window(c), osem)

    for b in range(2):
        do_chunk(b, *bufs[b], True)

    def super_step(s, carry):
        for b in range(2):
            do_chunk(2 * s + b, *bufs[b], False)
        return carry

    lax.fori_loop(1, N_OUTER // 2, super_step, 0)
    for b in range(2):
        rows_v, osem = bufs[b]
        pltpu.make_async_copy(rows_v, out_window(N_OUTER - 2 + b), osem).wait()


def kernel(x, table):
    out128 = _emb_lookup(x, table)
    return out128[:, :, :DIM]
